# Initial kernel scaffold; baseline (speedup 1.0000x reference)
import jax
import jax.numpy as jnp
from jax.experimental import pallas as pl


def kernel(x, edge_index, W1l, b1, W1r, W2l, b2, W2r, Wd1, bd1, Wd2, bd2):
    raise NotImplementedError("write your pallas kernel here")


# speedup vs baseline: 4.0263x; 4.0263x over previous
import jax
import jax.numpy as jnp
from jax import lax
from jax.experimental import pallas as pl
from jax.experimental.pallas import tpu as pltpu
from jax.experimental.pallas import tpu_sc as plsc

_N = 10000
_E = 320000
_F = 128
_NC = 2
_NS = 16
_NW = _NC * _NS
_EPT = _E // _NW
_CH = 80
_NCH = _EPT // _CH
_SEC = 25
_NSEC = _NCH // _SEC
_RPT = 624
_RLAST = _N - (_NS - 1) * _RPT
_RB = 2000
_GR = _N // _RB

_mesh = plsc.VectorSubcoreMesh(core_axis_name="c", subcore_axis_name="s")
_f32 = jnp.float32


def _seg_body(with_deg, *refs):
    if with_deg:
        (y, srcr, dstr, z128, zdeg, agg_o, deg_o,
         src_v, dst_v, rows_v, agg_sh, ones_v, deg_sh) = refs
    else:
        (y, srcr, dstr, z128, agg_o,
         src_v, dst_v, rows_v, agg_sh) = refs
    c = lax.axis_index("c")
    s = lax.axis_index("s")
    w = c * _NS + s

    @pl.when(s < _NS - 1)
    def _():
        pltpu.sync_copy(z128.at[pl.ds(0, _RPT)], agg_sh.at[pl.ds(s * _RPT, _RPT)])

    @pl.when(s == _NS - 1)
    def _():
        pltpu.sync_copy(z128, agg_sh.at[pl.ds((_NS - 1) * _RPT, _RLAST)])

    if with_deg:
        @pl.when(s == 0)
        def _():
            pltpu.sync_copy(zdeg, deg_sh)

        ones16 = jnp.ones((16,), _f32)
        for i in range(_CH // 16):
            ones_v[pl.ds(i * 16, 16)] = ones16
    plsc.subcore_barrier()

    def section(t, carry):
        pltpu.sync_copy(srcr.at[w * _NSEC + t], src_v)
        pltpu.sync_copy(dstr.at[w * _NSEC + t], dst_v)

        def chunk(j, c2):
            pltpu.sync_copy(y.at[src_v.at[j]], rows_v)
            pltpu.sync_copy(rows_v, agg_sh.at[dst_v.at[j]], add=True)
            if with_deg:
                pltpu.sync_copy(ones_v, deg_sh.at[dst_v.at[j]], add=True)
            return c2

        lax.fori_loop(0, _SEC, chunk, 0)
        return carry

    lax.fori_loop(0, _NSEC, section, 0)
    plsc.subcore_barrier()

    @pl.when(s < _NS - 1)
    def _():
        pltpu.sync_copy(agg_sh.at[pl.ds(s * _RPT, _RPT)],
                        agg_o.at[pl.ds(c * _N + s * _RPT, _RPT)])

    @pl.when(s == _NS - 1)
    def _():
        pltpu.sync_copy(agg_sh.at[pl.ds((_NS - 1) * _RPT, _RLAST)],
                        agg_o.at[pl.ds(c * _N + (_NS - 1) * _RPT, _RLAST)])

    if with_deg:
        @pl.when(s == 0)
        def _():
            pltpu.sync_copy(deg_sh, deg_o.at[c])


def _seg_sum(y, srcr, dstr, z128, with_deg=False):
    outs = [jax.ShapeDtypeStruct((2 * _N, _F), _f32)]
    scratch = [
        pltpu.VMEM((_SEC, _CH), jnp.int32),
        pltpu.VMEM((_SEC, _CH), jnp.int32),
        pltpu.VMEM((_CH, _F), _f32),
        pltpu.VMEM_SHARED((_N, _F), _f32),
    ]
    args = [y, srcr, dstr, z128]
    if with_deg:
        outs.append(jax.ShapeDtypeStruct((_NC, _N), _f32))
        scratch += [pltpu.VMEM((_CH,), _f32),
                    pltpu.VMEM_SHARED((_N,), _f32)]
        args.append(jnp.zeros((_N,), _f32))
    fn = pl.kernel(
        lambda *r: _seg_body(with_deg, *r),
        out_type=tuple(outs),
        mesh=_mesh,
        scratch_types=tuple(scratch),
    )
    return fn(*args)


def _dec_body(a_t, b_t, srcr, dstr, wv, out,
              src_v, dst_v, u_v, b_v, wv_v, out_v):
    c = lax.axis_index("c")
    s = lax.axis_index("s")
    w = c * _NS + s
    pltpu.sync_copy(wv, wv_v)
    w8 = [wv_v[pl.ds(b * 16, 16)] for b in range(_F // 16)]
    zero = jnp.zeros((16,), _f32)

    def section(t, carry):
        pltpu.sync_copy(srcr.at[w * _NSEC + t], src_v)
        pltpu.sync_copy(dstr.at[w * _NSEC + t], dst_v)
        base = (w * _NCH + t * _SEC) * _CH

        def chunk(j, c2):
            pltpu.sync_copy(a_t.at[src_v.at[j]], u_v)
            pltpu.sync_copy(b_t.at[dst_v.at[j]], b_v)
            for ee in range(_CH):
                acc = zero
                for b in range(_F // 16):
                    u = u_v[ee, pl.ds(b * 16, 16)] + b_v[ee, pl.ds(b * 16, 16)]
                    acc = acc + jnp.maximum(u, 0.0) * w8[b]
                out_v[ee, :] = acc
            pltpu.sync_copy(out_v, out.at[pl.ds(base + j * _CH, _CH)])
            return c2

        lax.fori_loop(0, _SEC, chunk, 0)
        return carry

    lax.fori_loop(0, _NSEC, section, 0)


def _decode(a_t, b_t, srcr, dstr, wv):
    fn = pl.kernel(
        _dec_body,
        out_type=jax.ShapeDtypeStruct((_E, 16), _f32),
        mesh=_mesh,
        scratch_types=(
            pltpu.VMEM((_SEC, _CH), jnp.int32),
            pltpu.VMEM((_SEC, _CH), jnp.int32),
            pltpu.VMEM((_CH, _F), _f32),
            pltpu.VMEM((_CH, _F), _f32),
            pltpu.VMEM((_F,), _f32),
            pltpu.VMEM((_CH, 16), _f32),
        ),
    )
    return fn(a_t, b_t, srcr, dstr, wv)


_EBE = 32000


def _esum_body(u_r, g_r, o_r):
    o_r[...] = jnp.dot(u_r[...], g_r[...], preferred_element_type=_f32)


def _edge_sum(u, g):
    return pl.pallas_call(
        _esum_body,
        grid=(_E // _EBE,),
        in_specs=[pl.BlockSpec((_EBE // 8, 128), lambda i: (i, 0)),
                  pl.BlockSpec((128, 8), lambda i: (0, 0))],
        out_specs=pl.BlockSpec((_EBE // 8, 8), lambda i: (i, 0)),
        out_shape=jax.ShapeDtypeStruct((_E // 8, 8), _f32),
    )(u, g)

def _row_spec():
    return pl.BlockSpec((_RB, _F), lambda i: (i, 0))


def _w_spec():
    return pl.BlockSpec((_F, _F), lambda i: (0, 0))


def _b_spec():
    return pl.BlockSpec((1, _F), lambda i: (0, 0))


def _enc_body(x_r, Wl_r, Wr_r, b_r, y_o, r_o):
    xb = x_r[...]
    y_o[...] = jnp.dot(xb, Wl_r[...], preferred_element_type=_f32)
    r_o[...] = jnp.dot(xb, Wr_r[...], preferred_element_type=_f32) + b_r[...]


def _encode(x, Wl, Wr, b):
    return pl.pallas_call(
        _enc_body,
        grid=(_GR,),
        in_specs=[_row_spec(), _w_spec(), _w_spec(), _b_spec()],
        out_specs=[_row_spec(), _row_spec()],
        out_shape=[jax.ShapeDtypeStruct((_N, _F), _f32)] * 2,
    )(x, Wl, Wr, b)


def _agg_specs():
    return [pl.BlockSpec((_RB, _F), lambda i: (i, 0)),
            pl.BlockSpec((_RB, _F), lambda i: (i + _GR, 0)),
            pl.BlockSpec((_RB, _NC), lambda i: (i, 0))]


def _node_z(agg0, agg1, dg, rb):
    deg = jnp.maximum(jnp.sum(dg[...], axis=1, keepdims=True), 1.0)
    return (agg0[...] + agg1[...]) / deg + rb[...]


def _mid_body(agg0, agg1, dg, rb, Wl_r, Wr_r, b_r, y_o, r_o):
    h = jnp.maximum(_node_z(agg0, agg1, dg, rb), 0.0)
    y_o[...] = jnp.dot(h, Wl_r[...], preferred_element_type=_f32)
    r_o[...] = jnp.dot(h, Wr_r[...], preferred_element_type=_f32) + b_r[...]


def _mid(aggp, degt, rb, Wl, Wr, b):
    return pl.pallas_call(
        _mid_body,
        grid=(_GR,),
        in_specs=_agg_specs() + [_row_spec(), _w_spec(), _w_spec(), _b_spec()],
        out_specs=[_row_spec(), _row_spec()],
        out_shape=[jax.ShapeDtypeStruct((_N, _F), _f32)] * 2,
    )(aggp, aggp, degt, rb, Wl, Wr, b)


def _fin_body(agg0, agg1, dg, rb, Wa_r, Wb_r, ba_r, a_o, b_o):
    z = _node_z(agg0, agg1, dg, rb)
    a_o[...] = jnp.dot(z, Wa_r[...], preferred_element_type=_f32) + ba_r[...]
    b_o[...] = jnp.dot(z, Wb_r[...], preferred_element_type=_f32)


def _fin(aggp, degt, rb, Wa, Wb, ba):
    return pl.pallas_call(
        _fin_body,
        grid=(_GR,),
        in_specs=_agg_specs() + [_row_spec(), _w_spec(), _w_spec(), _b_spec()],
        out_specs=[_row_spec(), _row_spec()],
        out_shape=[jax.ShapeDtypeStruct((_N, _F), _f32)] * 2,
    )(aggp, aggp, degt, rb, Wa, Wb, ba)


def kernel(x, edge_index, W1l, b1, W1r, W2l, b2, W2r, Wd1, bd1, Wd2, bd2):
    srcr = edge_index[0].reshape(_NW * _NSEC, _SEC, _CH)
    dstr = edge_index[1].reshape(_NW * _NSEC, _SEC, _CH)
    z128 = jnp.zeros((_RLAST, _F), _f32)

    y1, r1 = _encode(x, W1l, W1r, b1.reshape(1, _F))
    agg1, degp = _seg_sum(y1, srcr, dstr, z128, with_deg=True)
    degt = degp.T
    y2, r2 = _mid(agg1, degt, r1, W2l, W2r, b2.reshape(1, _F))
    agg2 = _seg_sum(y2, srcr, dstr, z128)[0]
    a_t, b_t = _fin(agg2, degt, r2, Wd1[:_F], Wd1[_F:], bd1.reshape(1, _F))
    e16 = _decode(a_t, b_t, srcr, dstr, Wd2.reshape(_F))
    gmat = jnp.repeat(jnp.eye(8, dtype=_f32), 16, axis=0)
    return _edge_sum(e16.reshape(_E // 8, _F), gmat).reshape(_E) + bd2[0]
